# Initial kernel scaffold; baseline (speedup 1.0000x reference)
import jax
import jax.numpy as jnp
from jax import lax
from jax.experimental import pallas as pl
from jax.experimental.pallas import tpu as pltpu
from jax.experimental.pallas import tpu_sc as plsc

VOCAB = 1000000
EMB = 64
B = 4096
F = 26
L = 20

NUM_WORKERS = 32
SEGS = B * F
SEGS_PER_W = SEGS // NUM_WORKERS
CHUNK = 32
ROWS = CHUNK * L
GATHER_SPLIT = 128
N_GATHER = ROWS // GATHER_SPLIT
N_CHUNKS = SEGS_PER_W // CHUNK
EV = EMB // 16


def _sc_body(x_hbm, table_hbm, out_hbm, idx_v, rows_v, out_v, sem):
    nc = 2
    wid = lax.axis_index("s") * nc + lax.axis_index("c")

    def chunk_body(it, carry):
        seg_base = wid * SEGS_PER_W + it * CHUNK
        idx_off = seg_base * L
        pltpu.sync_copy(x_hbm.at[pl.ds(idx_off, ROWS)], idx_v)
        cps = [
            pltpu.async_copy(
                table_hbm.at[idx_v.at[pl.ds(j * GATHER_SPLIT, GATHER_SPLIT)]],
                rows_v.at[pl.ds(j * GATHER_SPLIT, GATHER_SPLIT)],
                sem,
            )
            for j in range(N_GATHER)
        ]
        for c in cps:
            c.wait()

        def seg_body(s, carry2):
            row0 = s * L
            accs = [rows_v[row0, pl.ds(e * 16, 16)] for e in range(EV)]
            for l in range(1, L):
                for e in range(EV):
                    accs[e] = accs[e] + rows_v[row0 + l, pl.ds(e * 16, 16)]
            for e in range(EV):
                out_v[s, pl.ds(e * 16, 16)] = accs[e]
            return carry2

        lax.fori_loop(0, CHUNK, seg_body, 0)
        pltpu.sync_copy(out_v, out_hbm.at[pl.ds(seg_base, CHUNK)])
        return carry

    lax.fori_loop(0, N_CHUNKS, chunk_body, 0)


def kernel(x, table):
    x_flat = x.reshape(-1).astype(jnp.int32)
    mesh = plsc.VectorSubcoreMesh(core_axis_name="c", subcore_axis_name="s")
    out = pl.kernel(
        _sc_body,
        out_type=jax.ShapeDtypeStruct((SEGS, EMB), jnp.float32),
        mesh=mesh,
        scratch_types=[
            pltpu.VMEM((ROWS,), jnp.int32),
            pltpu.VMEM((ROWS, EMB), jnp.float32),
            pltpu.VMEM((CHUNK, EMB), jnp.float32),
            pltpu.SemaphoreType.DMA,
        ],
    )(x_flat, table)
    return out.reshape(B, F * EMB)


# speedup vs baseline: 2.3548x; 2.3548x over previous
import jax
import jax.numpy as jnp
from jax import lax
from jax.experimental import pallas as pl
from jax.experimental.pallas import tpu as pltpu
from jax.experimental.pallas import tpu_sc as plsc

VOCAB = 1000000
EMB = 64
B = 4096
F = 26
L = 20

NUM_WORKERS = 32
SEGS = B * F
SEGS_PER_W = SEGS // NUM_WORKERS
CHUNK = 32
ROWS = CHUNK * L
GATHER_SPLIT = 128
N_GATHER = ROWS // GATHER_SPLIT
N_CHUNKS = SEGS_PER_W // CHUNK
EV = EMB // 16


def _sc_body(x_hbm, table_hbm, out_hbm, idx_v, rows_v, out_v, sem):
    nc = 2
    wid = lax.axis_index("s") * nc + lax.axis_index("c")

    def chunk_body(it, carry):
        seg_base = wid * SEGS_PER_W + it * CHUNK
        idx_off = seg_base * L
        pltpu.sync_copy(x_hbm.at[pl.ds(idx_off, ROWS)], idx_v)
        cps = [
            pltpu.async_copy(
                table_hbm.at[idx_v.at[pl.ds(j * GATHER_SPLIT, GATHER_SPLIT)]],
                rows_v.at[pl.ds(j * GATHER_SPLIT, GATHER_SPLIT)],
                sem,
            )
            for j in range(N_GATHER)
        ]
        for c in cps:
            c.wait()

        def seg_body(s, carry2):
            row0 = s * L
            accs = [rows_v[row0, pl.ds(e * 16, 16)] for e in range(EV)]
            for l in range(1, L):
                for e in range(EV):
                    accs[e] = accs[e] + rows_v[row0 + l, pl.ds(e * 16, 16)]
            for e in range(EV):
                out_v[s, pl.ds(e * 16, 16)] = accs[e]
            return carry2

        lax.fori_loop(0, CHUNK, seg_body, 0)
        pltpu.sync_copy(out_v, out_hbm.at[pl.ds(seg_base, CHUNK)])
        return carry

    lax.fori_loop(0, N_CHUNKS, chunk_body, 0)


def kernel(x, table):
    x_flat = x.reshape(-1).astype(jnp.int32)
    mesh = plsc.VectorSubcoreMesh(core_axis_name="c", subcore_axis_name="s")
    out = pl.kernel(
        _sc_body,
        out_type=jax.ShapeDtypeStruct((SEGS, EMB), jnp.float32),
        mesh=mesh,
        scratch_types=[
            pltpu.VMEM((ROWS,), jnp.int32),
            pltpu.VMEM((ROWS, EMB), jnp.float32),
            pltpu.VMEM((CHUNK, EMB), jnp.float32),
            pltpu.SemaphoreType.DMA,
        ],
        compiler_params=pltpu.CompilerParams(use_tc_tiling_on_sc=False),
    )(x_flat, table)
    return out.reshape(B, F * EMB)


# speedup vs baseline: 2.9381x; 1.2477x over previous
import jax
import jax.numpy as jnp
from jax import lax
from jax.experimental import pallas as pl
from jax.experimental.pallas import tpu as pltpu
from jax.experimental.pallas import tpu_sc as plsc

VOCAB = 1000000
EMB = 64
B = 4096
F = 26
L = 20

NUM_WORKERS = 32
SEGS = B * F
SEGS_PER_W = SEGS // NUM_WORKERS
CHUNK = 32
ROWS = CHUNK * L
GATHER_SPLIT = 128
N_GATHER = ROWS // GATHER_SPLIT
N_CHUNKS = SEGS_PER_W // CHUNK
EV = EMB // 16


def _sc_body(x_hbm, table_hbm, out_hbm,
             idx0, idx1, rows0, rows1, out_v, sem0, sem1):
    nc = 2
    wid = lax.axis_index("s") * nc + lax.axis_index("c")

    def fire(itc, idx_v, rows_v, sem):
        idx_off = (wid * SEGS_PER_W + itc * CHUNK) * L
        pltpu.sync_copy(x_hbm.at[pl.ds(idx_off, ROWS)], idx_v)
        for j in range(N_GATHER):
            sl = pl.ds(j * GATHER_SPLIT, GATHER_SPLIT)
            pltpu.async_copy(table_hbm.at[idx_v.at[sl]], rows_v.at[sl], sem)

    def drain(idx_v, rows_v, sem):
        for j in range(N_GATHER):
            sl = pl.ds(j * GATHER_SPLIT, GATHER_SPLIT)
            pltpu.make_async_copy(
                table_hbm.at[idx_v.at[sl]], rows_v.at[sl], sem).wait()

    def reduce_store(itc, rows_v):
        @plsc.parallel_loop(0, CHUNK, unroll=2)
        def seg_body(s):
            row0 = s * L
            for e in range(EV):
                sl = pl.ds(e * 16, 16)
                a = rows_v[row0, sl]
                b = rows_v[row0 + 1, sl]
                for l in range(2, L, 2):
                    a = a + rows_v[row0 + l, sl]
                    b = b + rows_v[row0 + l + 1, sl]
                out_v[s, sl] = a + b

        seg_base = wid * SEGS_PER_W + itc * CHUNK
        pltpu.sync_copy(out_v, out_hbm.at[pl.ds(seg_base, CHUNK)])

    fire(0, idx0, rows0, sem0)

    def pair_body(p, carry):
        it0 = 2 * p
        it1 = 2 * p + 1
        it2 = 2 * p + 2
        fire(it1, idx1, rows1, sem1)
        drain(idx0, rows0, sem0)
        reduce_store(it0, rows0)

        @pl.when(it2 < N_CHUNKS)
        def _():
            fire(it2, idx0, rows0, sem0)

        drain(idx1, rows1, sem1)
        reduce_store(it1, rows1)
        return carry

    lax.fori_loop(0, N_CHUNKS // 2, pair_body, 0)


def kernel(x, table):
    x_flat = x.reshape(-1).astype(jnp.int32)
    mesh = plsc.VectorSubcoreMesh(core_axis_name="c", subcore_axis_name="s")
    out = pl.kernel(
        _sc_body,
        out_type=jax.ShapeDtypeStruct((SEGS, EMB), jnp.float32),
        mesh=mesh,
        scratch_types=[
            pltpu.VMEM((ROWS,), jnp.int32),
            pltpu.VMEM((ROWS,), jnp.int32),
            pltpu.VMEM((ROWS, EMB), jnp.float32),
            pltpu.VMEM((ROWS, EMB), jnp.float32),
            pltpu.VMEM((CHUNK, EMB), jnp.float32),
            pltpu.SemaphoreType.DMA,
            pltpu.SemaphoreType.DMA,
        ],
        compiler_params=pltpu.CompilerParams(use_tc_tiling_on_sc=False),
    )(x_flat, table)
    return out.reshape(B, F * EMB)


# speedup vs baseline: 3.0322x; 1.0320x over previous
import jax
import jax.numpy as jnp
from jax import lax
from jax.experimental import pallas as pl
from jax.experimental.pallas import tpu as pltpu
from jax.experimental.pallas import tpu_sc as plsc

VOCAB = 1000000
EMB = 64
B = 4096
F = 26
L = 20

NUM_WORKERS = 32
SEGS = B * F
SEGS_PER_W = SEGS // NUM_WORKERS
CHUNK = 32
ROWS = CHUNK * L
GATHER_SPLIT = 128
N_GATHER = ROWS // GATHER_SPLIT
N_CHUNKS = SEGS_PER_W // CHUNK
EV = EMB // 16
IDX_BLK = 8


def _sc_body(x_hbm, table_hbm, out_hbm,
             idx_v, rows0, rows1, outa, outb, sem0, sem1, semoa, semob):
    nc = 2
    wid = lax.axis_index("s") * nc + lax.axis_index("c")

    def sync_idx_block(blk):
        idx_off = (wid * SEGS_PER_W + blk * IDX_BLK * CHUNK) * L
        slot = (blk % 2) * (IDX_BLK * ROWS)
        pltpu.sync_copy(x_hbm.at[pl.ds(idx_off, IDX_BLK * ROWS)],
                        idx_v.at[pl.ds(slot, IDX_BLK * ROWS)])

    def fire(itc, rows_v, sem):
        base = (itc % (2 * IDX_BLK)) * ROWS
        for j in range(N_GATHER):
            isl = pl.ds(base + j * GATHER_SPLIT, GATHER_SPLIT)
            rsl = pl.ds(j * GATHER_SPLIT, GATHER_SPLIT)
            pltpu.async_copy(table_hbm.at[idx_v.at[isl]], rows_v.at[rsl], sem)

    def drain(itc, rows_v, sem):
        base = (itc % (2 * IDX_BLK)) * ROWS
        for j in range(N_GATHER):
            isl = pl.ds(base + j * GATHER_SPLIT, GATHER_SPLIT)
            rsl = pl.ds(j * GATHER_SPLIT, GATHER_SPLIT)
            pltpu.make_async_copy(
                table_hbm.at[idx_v.at[isl]], rows_v.at[rsl], sem).wait()

    def reduce(rows_v, out_v):
        @plsc.parallel_loop(0, CHUNK, unroll=4)
        def seg_body(s):
            row0 = s * L
            for e in range(EV):
                sl = pl.ds(e * 16, 16)
                a = rows_v[row0, sl]
                b = rows_v[row0 + 1, sl]
                for l in range(2, L, 2):
                    a = a + rows_v[row0 + l, sl]
                    b = b + rows_v[row0 + l + 1, sl]
                out_v[s, sl] = a + b

    def out_start(itc, out_v, semo):
        seg_base = wid * SEGS_PER_W + itc * CHUNK
        pltpu.async_copy(out_v, out_hbm.at[pl.ds(seg_base, CHUNK)], semo)

    def out_wait(out_v, semo):
        pltpu.make_async_copy(out_v, out_hbm.at[pl.ds(0, CHUNK)], semo).wait()

    sync_idx_block(0)
    fire(0, rows0, sem0)

    def pair_body(p, carry):
        it0 = 2 * p
        it1 = 2 * p + 1
        it2 = 2 * p + 2

        fire(it1, rows1, sem1)

        @pl.when(p > 0)
        def _():
            out_wait(outa, semoa)

        drain(it0, rows0, sem0)
        reduce(rows0, outa)
        out_start(it0, outa, semoa)

        @pl.when(jnp.logical_and(it2 % IDX_BLK == 0, it2 < N_CHUNKS))
        def _():
            sync_idx_block(it2 // IDX_BLK)

        @pl.when(it2 < N_CHUNKS)
        def _():
            fire(it2, rows0, sem0)

        @pl.when(p > 0)
        def _():
            out_wait(outb, semob)

        drain(it1, rows1, sem1)
        reduce(rows1, outb)
        out_start(it1, outb, semob)
        return carry

    lax.fori_loop(0, N_CHUNKS // 2, pair_body, 0)
    out_wait(outa, semoa)
    out_wait(outb, semob)


def kernel(x, table):
    x_flat = x.reshape(-1).astype(jnp.int32)
    mesh = plsc.VectorSubcoreMesh(core_axis_name="c", subcore_axis_name="s")
    out = pl.kernel(
        _sc_body,
        out_type=jax.ShapeDtypeStruct((SEGS, EMB), jnp.float32),
        mesh=mesh,
        scratch_types=[
            pltpu.VMEM((2 * IDX_BLK * ROWS,), jnp.int32),
            pltpu.VMEM((ROWS, EMB), jnp.float32),
            pltpu.VMEM((ROWS, EMB), jnp.float32),
            pltpu.VMEM((CHUNK, EMB), jnp.float32),
            pltpu.VMEM((CHUNK, EMB), jnp.float32),
            pltpu.SemaphoreType.DMA,
            pltpu.SemaphoreType.DMA,
            pltpu.SemaphoreType.DMA,
            pltpu.SemaphoreType.DMA,
        ],
        compiler_params=pltpu.CompilerParams(use_tc_tiling_on_sc=False),
    )(x_flat, table)
    return out.reshape(B, F * EMB)
